# baseline (speedup 1.0000x reference)
import functools

import jax
import jax.numpy as jnp
from jax import lax
from jax.experimental import pallas as pl
from jax.experimental.pallas import tpu as pltpu
from jax.experimental.pallas import tpu_sc as plsc

NC = 2
NS = 16
NW = NC * NS
L = 16


def _build_sc_interp(n_points, n_frames, g):
    ppw = n_points // NW
    B = 1024
    nchunk = ppw // B
    ng = B // L
    nidx = 16 * B
    row = 128
    nrow = nidx // row

    g2 = g * g
    g3 = g * g * g
    scale = float(g - 1)
    tscale = float(n_frames - 1)
    roffs = [fr * g3 + dx * g2 for fr in (0, 1) for dx in (0, 1)]

    mesh = plsc.VectorSubcoreMesh(core_axis_name="c", subcore_axis_name="s")

    buf_types = [
        pltpu.VMEM((4 * B,), jnp.float32),
        pltpu.VMEM((nidx,), jnp.int32),
        pltpu.VMEM((nidx,), jnp.float32),
        pltpu.VMEM((nidx,), jnp.float32),
        pltpu.VMEM((4 * B,), jnp.float32),
        pltpu.SemaphoreType.DMA,
    ]

    @functools.partial(
        pl.kernel,
        out_type=jax.ShapeDtypeStruct((2 * n_points,), jnp.float32),
        mesh=mesh,
        compiler_params=pltpu.CompilerParams(needs_layout_passes=False),
        scratch_types=buf_types + buf_types + [
            pltpu.VMEM((2 * B,), jnp.float32),
        ],
    )
    def sc_interp(qp_hbm, rho_hbm, tt_hbm, out_hbm,
                  qb0, ix0, rv0, tv0, fb0, sem0,
                  qb1, ix1, rv1, tv1, fb1, sem1, outb):
        wid = lax.axis_index("s") * NC + lax.axis_index("c")
        iota = lax.iota(jnp.int32, L)
        bufs = ((qb0, ix0, rv0, tv0, fb0, sem0),
                (qb1, ix1, rv1, tv1, fb1, sem1))

        def phase_a(ci, par):
            qb, ixb, rvals, tvals, fb, sem = bufs[par]
            base = wid * ppw + ci * B
            pltpu.sync_copy(qp_hbm.at[pl.ds(base * 4, 4 * B)], qb)

            def group_idx(gi, _):
                rows4 = (gi * L + iota) * 4
                x = plsc.load_gather(qb, [rows4])
                y = plsc.load_gather(qb, [rows4 + 1])
                z = plsc.load_gather(qb, [rows4 + 2])
                t = plsc.load_gather(qb, [rows4 + 3])
                xg = x * scale
                yg = y * scale
                zg = z * scale
                tg = t * tscale
                xi = jnp.minimum(jnp.maximum(xg.astype(jnp.int32), 0), g - 2)
                yi = jnp.minimum(jnp.maximum(yg.astype(jnp.int32), 0), g - 2)
                zi = jnp.minimum(jnp.maximum(zg.astype(jnp.int32), 0), g - 2)
                fi = jnp.minimum(jnp.maximum(tg.astype(jnp.int32), 0),
                                 n_frames - 2)
                fb[pl.ds(gi * L, L)] = xg - xi.astype(jnp.float32)
                fb[pl.ds(B + gi * L, L)] = yg - yi.astype(jnp.float32)
                fb[pl.ds(2 * B + gi * L, L)] = zg - zi.astype(jnp.float32)
                fb[pl.ds(3 * B + gi * L, L)] = tg - fi.astype(jnp.float32)
                base0 = fi * g3 + xi * g2 + yi * g + zi
                ppos = (gi * L + iota) * 16
                for k in range(4):
                    qbase = (base0 + roffs[k]) * 4
                    for l in range(4):
                        plsc.store_scatter(ixb, [ppos + (k * 4 + l)],
                                           qbase + l)
                return 0

            lax.fori_loop(0, ng, group_idx, 0)

            def fire_row(j, _):
                sl = pl.ds(j * row, row)
                pltpu.async_copy(rho_hbm.at[ixb.at[sl]], rvals.at[sl], sem)
                pltpu.async_copy(tt_hbm.at[ixb.at[sl]], tvals.at[sl], sem)
                return 0

            lax.fori_loop(0, nrow, fire_row, 0)

        def phase_c(ci, par):
            qb, ixb, rvals, tvals, fb, sem = bufs[par]
            base = wid * ppw + ci * B
            pltpu.make_async_copy(rho_hbm.at[pl.ds(0, nidx)], rvals,
                                  sem).wait()
            pltpu.make_async_copy(tt_hbm.at[pl.ds(0, nidx)], tvals,
                                  sem).wait()

            def group_out(gi, _):
                fx = fb[pl.ds(gi * L, L)]
                fy = fb[pl.ds(B + gi * L, L)]
                fz = fb[pl.ds(2 * B + gi * L, L)]
                ft = fb[pl.ds(3 * B + gi * L, L)]
                ppos = (gi * L + iota) * 16

                def tri(vals, fr):
                    c = [plsc.load_gather(
                            vals, [ppos + ((fr * 2 + dx) * 4 + dy * 2 + dz)])
                         for dx in (0, 1) for dy in (0, 1) for dz in (0, 1)]
                    c00 = c[0] + fz * (c[1] - c[0])
                    c01 = c[2] + fz * (c[3] - c[2])
                    c10 = c[4] + fz * (c[5] - c[4])
                    c11 = c[6] + fz * (c[7] - c[6])
                    c0 = c00 + fy * (c01 - c00)
                    c1 = c10 + fy * (c11 - c10)
                    return c0 + fx * (c1 - c0)

                r1 = tri(rvals, 0)
                r2 = tri(rvals, 1)
                t1 = tri(tvals, 0)
                t2 = tri(tvals, 1)
                rho = r1 + ft * (r2 - r1)
                tmp = t1 + ft * (t2 - t1)
                pos = (gi * L + iota) * 2
                plsc.store_scatter(outb, [pos], rho)
                plsc.store_scatter(outb, [pos + 1], tmp)
                return 0

            lax.fori_loop(0, ng, group_out, 0)
            pltpu.sync_copy(outb, out_hbm.at[pl.ds(base * 2, 2 * B)])

        phase_a(0, 0)

        def pipe_body(k, _):
            ci0 = 2 * k
            phase_a(ci0 + 1, 1)
            phase_c(ci0, 0)

            @pl.when(ci0 + 2 < nchunk)
            def _():
                phase_a(ci0 + 2, 0)

            phase_c(ci0 + 1, 1)
            return 0

        lax.fori_loop(0, nchunk // 2, pipe_body, 0)

    return sc_interp


def _quad_table(frames, g):
    flat = frames.reshape(-1)
    v = flat.shape[0]
    padded = jnp.pad(flat, (0, g + 2))
    cols = [padded[k:k + v] for k in (0, 1, g, g + 1)]
    return jnp.stack(cols, axis=1).reshape(-1)


def kernel(query_points, frames_rho, frames_T, log_abs, vol_c):
    n = query_points.shape[0]
    nf, g = frames_rho.shape[0], frames_rho.shape[1]
    sc_interp = _build_sc_interp(n, nf, g)
    flat = sc_interp(query_points.reshape(-1), _quad_table(frames_rho, g),
                     _quad_table(frames_T, g))
    return (flat.reshape(n, 2), log_abs, vol_c)


# speedup vs baseline: 12.5919x; 12.5919x over previous
import functools

import jax
import jax.numpy as jnp
from jax import lax
from jax.experimental import pallas as pl
from jax.experimental.pallas import tpu as pltpu
from jax.experimental.pallas import tpu_sc as plsc

NC = 2
NS = 16
NW = NC * NS
L = 16


def _build_sc_interp(n_points, n_frames, g):
    ppw = n_points // NW
    B = 1024
    nchunk = ppw // B
    ng = B // L
    nidx = 16 * B
    row = 128
    nrow = nidx // row

    g2 = g * g
    g3 = g * g * g
    scale = float(g - 1)
    tscale = float(n_frames - 1)
    qoffs = [fr * g3 + dx * g2 + dy * g
             for fr in (0, 1) for dx in (0, 1) for dy in (0, 1)]

    mesh = plsc.VectorSubcoreMesh(core_axis_name="c", subcore_axis_name="s")

    buf_types = [
        pltpu.VMEM((4 * B,), jnp.float32),
        pltpu.VMEM((nidx,), jnp.int32),
        pltpu.VMEM((nidx,), jnp.float32),
        pltpu.VMEM((nidx,), jnp.float32),
        pltpu.VMEM((4 * B,), jnp.float32),
        pltpu.SemaphoreType.DMA,
    ]

    @functools.partial(
        pl.kernel,
        out_type=jax.ShapeDtypeStruct((2 * n_points,), jnp.float32),
        mesh=mesh,
        compiler_params=pltpu.CompilerParams(needs_layout_passes=False),
        scratch_types=buf_types + buf_types + [
            pltpu.VMEM((2 * B,), jnp.float32),
        ],
    )
    def sc_interp(qp_hbm, rho_hbm, tt_hbm, out_hbm,
                  qb0, ix0, rv0, tv0, fb0, sem0,
                  qb1, ix1, rv1, tv1, fb1, sem1, outb):
        wid = lax.axis_index("s") * NC + lax.axis_index("c")
        iota = lax.iota(jnp.int32, L)
        bufs = ((qb0, ix0, rv0, tv0, fb0, sem0),
                (qb1, ix1, rv1, tv1, fb1, sem1))

        def phase_a(ci, par):
            qb, ixb, rvals, tvals, fb, sem = bufs[par]
            base = wid * ppw + ci * B
            pltpu.sync_copy(qp_hbm.at[pl.ds(base * 4, 4 * B)], qb)

            def group_idx(gi, _):
                rows4 = (gi * L + iota) * 4
                x = plsc.load_gather(qb, [rows4])
                y = plsc.load_gather(qb, [rows4 + 1])
                z = plsc.load_gather(qb, [rows4 + 2])
                t = plsc.load_gather(qb, [rows4 + 3])
                xg = x * scale
                yg = y * scale
                zg = z * scale
                tg = t * tscale
                xi = jnp.minimum(jnp.maximum(xg.astype(jnp.int32), 0), g - 2)
                yi = jnp.minimum(jnp.maximum(yg.astype(jnp.int32), 0), g - 2)
                zi = jnp.minimum(jnp.maximum(zg.astype(jnp.int32), 0), g - 2)
                fi = jnp.minimum(jnp.maximum(tg.astype(jnp.int32), 0),
                                 n_frames - 2)
                fb[pl.ds(gi * L, L)] = xg - xi.astype(jnp.float32)
                fb[pl.ds(B + gi * L, L)] = yg - yi.astype(jnp.float32)
                fb[pl.ds(2 * B + gi * L, L)] = zg - zi.astype(jnp.float32)
                fb[pl.ds(3 * B + gi * L, L)] = tg - fi.astype(jnp.float32)
                base0 = fi * g3 + xi * g2 + yi * g + zi
                ppos = (gi * L + iota) * 16
                for q in range(8):
                    for dz in (0, 1):
                        plsc.store_scatter(ixb, [ppos + (q * 2 + dz)],
                                           base0 + (qoffs[q] + dz))
                return 0

            lax.fori_loop(0, ng, group_idx, 0)

            def fire_row(j, _):
                sl = pl.ds(j * row, row)
                pltpu.async_copy(rho_hbm.at[ixb.at[sl]], rvals.at[sl], sem)
                pltpu.async_copy(tt_hbm.at[ixb.at[sl]], tvals.at[sl], sem)
                return 0

            lax.fori_loop(0, nrow, fire_row, 0)

        def phase_c(ci, par):
            qb, ixb, rvals, tvals, fb, sem = bufs[par]
            base = wid * ppw + ci * B
            pltpu.make_async_copy(rho_hbm.at[pl.ds(0, nidx)], rvals,
                                  sem).wait()
            pltpu.make_async_copy(tt_hbm.at[pl.ds(0, nidx)], tvals,
                                  sem).wait()

            def group_out(gi, _):
                fx = fb[pl.ds(gi * L, L)]
                fy = fb[pl.ds(B + gi * L, L)]
                fz = fb[pl.ds(2 * B + gi * L, L)]
                ft = fb[pl.ds(3 * B + gi * L, L)]
                ppos = (gi * L + iota) * 16

                def tri(vals, fr):
                    c = [plsc.load_gather(
                            vals,
                            [ppos + ((fr * 4 + dx * 2 + dy) * 2 + dz)])
                         for dx in (0, 1) for dy in (0, 1) for dz in (0, 1)]
                    c00 = c[0] + fz * (c[1] - c[0])
                    c01 = c[2] + fz * (c[3] - c[2])
                    c10 = c[4] + fz * (c[5] - c[4])
                    c11 = c[6] + fz * (c[7] - c[6])
                    c0 = c00 + fy * (c01 - c00)
                    c1 = c10 + fy * (c11 - c10)
                    return c0 + fx * (c1 - c0)

                r1 = tri(rvals, 0)
                r2 = tri(rvals, 1)
                t1 = tri(tvals, 0)
                t2 = tri(tvals, 1)
                rho = r1 + ft * (r2 - r1)
                tmp = t1 + ft * (t2 - t1)
                pos = (gi * L + iota) * 2
                plsc.store_scatter(outb, [pos], rho)
                plsc.store_scatter(outb, [pos + 1], tmp)
                return 0

            lax.fori_loop(0, ng, group_out, 0)
            pltpu.sync_copy(outb, out_hbm.at[pl.ds(base * 2, 2 * B)])

        phase_a(0, 0)

        def pipe_body(k, _):
            ci0 = 2 * k
            phase_a(ci0 + 1, 1)
            phase_c(ci0, 0)

            @pl.when(ci0 + 2 < nchunk)
            def _():
                phase_a(ci0 + 2, 0)

            phase_c(ci0 + 1, 1)
            return 0

        lax.fori_loop(0, nchunk // 2, pipe_body, 0)

    return sc_interp


def kernel(query_points, frames_rho, frames_T, log_abs, vol_c):
    n = query_points.shape[0]
    nf, g = frames_rho.shape[0], frames_rho.shape[1]
    sc_interp = _build_sc_interp(n, nf, g)
    flat = sc_interp(query_points.reshape(-1), frames_rho.reshape(-1),
                     frames_T.reshape(-1))
    return (flat.reshape(n, 2), log_abs, vol_c)


# speedup vs baseline: 15.7911x; 1.2541x over previous
import functools

import jax
import jax.numpy as jnp
from jax import lax
from jax.experimental import pallas as pl
from jax.experimental.pallas import tpu as pltpu
from jax.experimental.pallas import tpu_sc as plsc

NC = 2
NS = 16
NW = NC * NS
L = 16


def _build_sc_interp(n_points, n_frames, g):
    ppw = n_points // NW
    B = 1024
    nchunk = ppw // B
    ng = B // L
    nidx = 16 * B
    row = 128
    nrow = nidx // row

    g2 = g * g
    g3 = g * g * g
    scale = float(g - 1)
    tscale = float(n_frames - 1)
    qoffs = [fr * g3 + dx * g2 + dy * g
             for fr in (0, 1) for dx in (0, 1) for dy in (0, 1)]

    mesh = plsc.VectorSubcoreMesh(core_axis_name="c", subcore_axis_name="s")

    buf_types = [
        pltpu.VMEM((4 * B,), jnp.float32),
        pltpu.VMEM((nidx,), jnp.int32),
        pltpu.VMEM((nidx,), jnp.float32),
        pltpu.VMEM((nidx,), jnp.float32),
        pltpu.VMEM((4 * B,), jnp.float32),
        pltpu.SemaphoreType.DMA,
    ]

    @functools.partial(
        pl.kernel,
        out_type=(jax.ShapeDtypeStruct((n_points,), jnp.float32),
                  jax.ShapeDtypeStruct((n_points,), jnp.float32)),
        mesh=mesh,
        compiler_params=pltpu.CompilerParams(needs_layout_passes=False),
        scratch_types=buf_types + buf_types + [
            pltpu.VMEM((B,), jnp.float32),
            pltpu.VMEM((B,), jnp.float32),
        ],
    )
    def sc_interp(qp_hbm, rho_hbm, tt_hbm, outr_hbm, outt_hbm,
                  qb0, ix0, rv0, tv0, fb0, sem0,
                  qb1, ix1, rv1, tv1, fb1, sem1, outbr, outbt):
        wid = lax.axis_index("s") * NC + lax.axis_index("c")
        iota = lax.iota(jnp.int32, L)
        bufs = ((qb0, ix0, rv0, tv0, fb0, sem0),
                (qb1, ix1, rv1, tv1, fb1, sem1))

        def phase_a(ci, par):
            qb, ixb, rvals, tvals, fb, sem = bufs[par]
            base = wid * ppw + ci * B
            pltpu.sync_copy(qp_hbm.at[pl.ds(base * 4, 4 * B)], qb)

            def group_idx(gi, _):
                rows4 = (gi * L + iota) * 4
                x = plsc.load_gather(qb, [rows4])
                y = plsc.load_gather(qb, [rows4 + 1])
                z = plsc.load_gather(qb, [rows4 + 2])
                t = plsc.load_gather(qb, [rows4 + 3])
                xg = x * scale
                yg = y * scale
                zg = z * scale
                tg = t * tscale
                xi = jnp.minimum(jnp.maximum(xg.astype(jnp.int32), 0), g - 2)
                yi = jnp.minimum(jnp.maximum(yg.astype(jnp.int32), 0), g - 2)
                zi = jnp.minimum(jnp.maximum(zg.astype(jnp.int32), 0), g - 2)
                fi = jnp.minimum(jnp.maximum(tg.astype(jnp.int32), 0),
                                 n_frames - 2)
                fb[pl.ds(gi * L, L)] = xg - xi.astype(jnp.float32)
                fb[pl.ds(B + gi * L, L)] = yg - yi.astype(jnp.float32)
                fb[pl.ds(2 * B + gi * L, L)] = zg - zi.astype(jnp.float32)
                fb[pl.ds(3 * B + gi * L, L)] = tg - fi.astype(jnp.float32)
                base0 = fi * g3 + xi * g2 + yi * g + zi
                ppos = (gi * L + iota) * 16
                for q in range(8):
                    for dz in (0, 1):
                        plsc.store_scatter(ixb, [ppos + (q * 2 + dz)],
                                           base0 + (qoffs[q] + dz))
                return 0

            lax.fori_loop(0, ng, group_idx, 0)

            def fire_row(j, _):
                sl = pl.ds(j * row, row)
                pltpu.async_copy(rho_hbm.at[ixb.at[sl]], rvals.at[sl], sem)
                pltpu.async_copy(tt_hbm.at[ixb.at[sl]], tvals.at[sl], sem)
                return 0

            lax.fori_loop(0, nrow, fire_row, 0)

        def phase_c(ci, par):
            qb, ixb, rvals, tvals, fb, sem = bufs[par]
            base = wid * ppw + ci * B
            pltpu.make_async_copy(rho_hbm.at[pl.ds(0, nidx)], rvals,
                                  sem).wait()
            pltpu.make_async_copy(tt_hbm.at[pl.ds(0, nidx)], tvals,
                                  sem).wait()

            def group_out(gi, _):
                fx = fb[pl.ds(gi * L, L)]
                fy = fb[pl.ds(B + gi * L, L)]
                fz = fb[pl.ds(2 * B + gi * L, L)]
                ft = fb[pl.ds(3 * B + gi * L, L)]
                ppos = (gi * L + iota) * 16

                def tri(vals, fr):
                    c = [plsc.load_gather(
                            vals,
                            [ppos + ((fr * 4 + dx * 2 + dy) * 2 + dz)])
                         for dx in (0, 1) for dy in (0, 1) for dz in (0, 1)]
                    c00 = c[0] + fz * (c[1] - c[0])
                    c01 = c[2] + fz * (c[3] - c[2])
                    c10 = c[4] + fz * (c[5] - c[4])
                    c11 = c[6] + fz * (c[7] - c[6])
                    c0 = c00 + fy * (c01 - c00)
                    c1 = c10 + fy * (c11 - c10)
                    return c0 + fx * (c1 - c0)

                r1 = tri(rvals, 0)
                r2 = tri(rvals, 1)
                t1 = tri(tvals, 0)
                t2 = tri(tvals, 1)
                rho = r1 + ft * (r2 - r1)
                tmp = t1 + ft * (t2 - t1)
                outbr[pl.ds(gi * L, L)] = rho
                outbt[pl.ds(gi * L, L)] = tmp
                return 0

            lax.fori_loop(0, ng, group_out, 0)
            pltpu.sync_copy(outbr, outr_hbm.at[pl.ds(base, B)])
            pltpu.sync_copy(outbt, outt_hbm.at[pl.ds(base, B)])

        phase_a(0, 0)

        def pipe_body(k, _):
            ci0 = 2 * k
            phase_a(ci0 + 1, 1)
            phase_c(ci0, 0)

            @pl.when(ci0 + 2 < nchunk)
            def _():
                phase_a(ci0 + 2, 0)

            phase_c(ci0 + 1, 1)
            return 0

        lax.fori_loop(0, nchunk // 2, pipe_body, 0)

    return sc_interp


def kernel(query_points, frames_rho, frames_T, log_abs, vol_c):
    n = query_points.shape[0]
    nf, g = frames_rho.shape[0], frames_rho.shape[1]
    sc_interp = _build_sc_interp(n, nf, g)
    rho_o, temp_o = sc_interp(query_points.reshape(-1),
                              frames_rho.reshape(-1),
                              frames_T.reshape(-1))
    return (jnp.stack((rho_o, temp_o), axis=-1), log_abs, vol_c)
